# initial kernel scaffold (unmeasured)
import jax
import jax.numpy as jnp
from jax import lax
from jax.experimental import pallas as pl
from jax.experimental.pallas import tpu as pltpu

N_DEV = 8
M = 4096
K = 4096
N_OUT = 8192
M_PER = M // N_DEV
K_PER = K // N_DEV


def kernel(x, w_mat):
    assert x.shape == (M, K_PER), x.shape
    assert w_mat.shape == (K, N_OUT), w_mat.shape

    def body(x_ref, w_ref, out_ref,
             gathered, wbuf, amax_send, amax_recv,
             a2a_send_sems, a2a_recv_sems,
             amax_send_sems, amax_recv_sems, w_sems):
        my = lax.axis_index("i")

        amax_recv[...] = jnp.zeros_like(amax_recv)

        bar = pltpu.get_barrier_semaphore()
        for j in range(N_DEV):
            @pl.when(j != my)
            def _():
                pl.semaphore_signal(
                    bar, inc=1, device_id=(j,),
                    device_id_type=pl.DeviceIdType.MESH,
                )
        pl.semaphore_wait(bar, N_DEV - 1)

        def a2a_send_desc(j):
            return pltpu.make_async_remote_copy(
                src_ref=x_ref.at[pl.ds(j * M_PER, M_PER), :],
                dst_ref=gathered.at[my],
                send_sem=a2a_send_sems.at[j],
                recv_sem=a2a_recv_sems.at[my],
                device_id=(j,),
                device_id_type=pl.DeviceIdType.MESH,
            )

        def a2a_recv_desc(k):
            return pltpu.make_async_remote_copy(
                src_ref=gathered.at[k],
                dst_ref=gathered.at[k],
                send_sem=a2a_send_sems.at[k],
                recv_sem=a2a_recv_sems.at[k],
                device_id=(k,),
                device_id_type=pl.DeviceIdType.MESH,
            )

        for j in range(N_DEV):
            @pl.when(j != my)
            def _():
                a2a_send_desc(j).start()

        def w_desc(k, slot):
            return pltpu.make_async_copy(
                w_ref.at[pl.ds(k * K_PER, K_PER), :],
                wbuf.at[slot],
                w_sems.at[slot],
            )

        w_desc(0, 0).start()
        for k in range(N_DEV):
            @pl.when(k == my)
            def _():
                gathered[k] = x_ref[k * M_PER:(k + 1) * M_PER, :]

            @pl.when(k != my)
            def _():
                a2a_recv_desc(k).wait_recv()

            if k + 1 < N_DEV:
                w_desc(k + 1, (k + 1) % 2).start()
            w_desc(k, k % 2).wait()
            prod = jnp.dot(gathered[k], wbuf[k % 2],
                           preferred_element_type=jnp.float32)
            if k == 0:
                out_ref[...] = prod
            else:
                out_ref[...] = out_ref[...] + prod

        amax_local = jnp.max(jnp.abs(out_ref[...]))
        amax_send[...] = jnp.full((8, 128), amax_local, jnp.float32)

        def amax_send_desc(j):
            return pltpu.make_async_remote_copy(
                src_ref=amax_send,
                dst_ref=amax_recv.at[my],
                send_sem=amax_send_sems.at[j],
                recv_sem=amax_recv_sems.at[my],
                device_id=(j,),
                device_id_type=pl.DeviceIdType.MESH,
            )

        def amax_recv_desc(k):
            return pltpu.make_async_remote_copy(
                src_ref=amax_send,
                dst_ref=amax_recv.at[k],
                send_sem=amax_send_sems.at[k],
                recv_sem=amax_recv_sems.at[k],
                device_id=(k,),
                device_id_type=pl.DeviceIdType.MESH,
            )

        for j in range(N_DEV):
            @pl.when(j != my)
            def _():
                amax_send_desc(j).start()
        for k in range(N_DEV):
            @pl.when(k != my)
            def _():
                amax_recv_desc(k).wait_recv()

        g_amax = jnp.maximum(jnp.max(amax_recv[...]), amax_local)

        scale = g_amax / 127.0
        y = out_ref[...]
        q = jnp.clip(jnp.round(y / scale), -127.0, 127.0)
        out_ref[...] = q * scale

        for j in range(N_DEV):
            @pl.when(j != my)
            def _():
                a2a_send_desc(j).wait_send()
                amax_send_desc(j).wait_send()

    return pl.pallas_call(
        body,
        out_shape=jax.ShapeDtypeStruct((M_PER, N_OUT), jnp.float32),
        in_specs=[
            pl.BlockSpec(memory_space=pltpu.VMEM),
            pl.BlockSpec(memory_space=pltpu.ANY),
        ],
        out_specs=pl.BlockSpec(memory_space=pltpu.VMEM),
        scratch_shapes=[
            pltpu.VMEM((N_DEV, M_PER, K_PER), jnp.float32),
            pltpu.VMEM((2, K_PER, N_OUT), jnp.float32),
            pltpu.VMEM((8, 128), jnp.float32),
            pltpu.VMEM((N_DEV, 8, 128), jnp.float32),
            pltpu.SemaphoreType.DMA((N_DEV,)),
            pltpu.SemaphoreType.DMA((N_DEV,)),
            pltpu.SemaphoreType.DMA((N_DEV,)),
            pltpu.SemaphoreType.DMA((N_DEV,)),
            pltpu.SemaphoreType.DMA((2,)),
        ],
        compiler_params=pltpu.CompilerParams(collective_id=0),
    )(x, w_mat)


# baseline (device time: 160382 ns/iter reference)
import jax
import jax.numpy as jnp
from jax import lax
from jax.experimental import pallas as pl
from jax.experimental.pallas import tpu as pltpu

N_DEV = 8
M = 4096
K = 4096
N_OUT = 8192
M_PER = M // N_DEV
K_PER = K // N_DEV
N_CHUNK = 2048
NCH = N_OUT // N_CHUNK


def kernel(x, w_mat):
    assert x.shape == (M, K_PER), x.shape
    assert w_mat.shape == (K, N_OUT), w_mat.shape

    def body(x_ref, w_ref, out_ref,
             gathered, wbuf, amax_send, amax_recv,
             a2a_send_sems, a2a_recv_sems,
             amax_send_sems, amax_recv_sems, w_sems, local_sem):
        my = lax.axis_index("i")

        amax_recv[...] = jnp.zeros_like(amax_recv)

        bar = pltpu.get_barrier_semaphore()
        for j in range(N_DEV):
            @pl.when(j != my)
            def _():
                pl.semaphore_signal(
                    bar, inc=1, device_id=(j,),
                    device_id_type=pl.DeviceIdType.MESH,
                )
        pl.semaphore_wait(bar, N_DEV - 1)

        def a2a_send_desc(j):
            return pltpu.make_async_remote_copy(
                src_ref=x_ref.at[pl.ds(j * M_PER, M_PER), :],
                dst_ref=gathered.at[my],
                send_sem=a2a_send_sems.at[j],
                recv_sem=a2a_recv_sems.at[my],
                device_id=(j,),
                device_id_type=pl.DeviceIdType.MESH,
            )

        def a2a_recv_desc(k):
            return pltpu.make_async_remote_copy(
                src_ref=gathered.at[k],
                dst_ref=gathered.at[k],
                send_sem=a2a_send_sems.at[k],
                recv_sem=a2a_recv_sems.at[k],
                device_id=(k,),
                device_id_type=pl.DeviceIdType.MESH,
            )

        for j in range(N_DEV):
            @pl.when(j != my)
            def _():
                a2a_send_desc(j).start()

        def local_desc():
            return pltpu.make_async_copy(
                x_ref.at[pl.ds(my * M_PER, M_PER), :],
                gathered.at[my],
                local_sem,
            )

        local_desc().start()

        NW = N_DEV * NCH

        def w_desc(widx, slot):
            k, h = widx // NCH, widx % NCH
            return pltpu.make_async_copy(
                w_ref.at[pl.ds(k * K_PER, K_PER),
                         pl.ds(h * N_CHUNK, N_CHUNK)],
                wbuf.at[slot],
                w_sems.at[slot],
            )

        w_desc(0, 0).start()
        for widx in range(NW):
            k, h = widx // NCH, widx % NCH
            if h == 0:
                @pl.when(k == my)
                def _():
                    local_desc().wait()

                @pl.when(k != my)
                def _():
                    a2a_recv_desc(k).wait_recv()

            if widx + 1 < NW:
                w_desc(widx + 1, (widx + 1) % 2).start()
            w_desc(widx, widx % 2).wait()
            prod = jnp.dot(gathered[k], wbuf[widx % 2],
                           preferred_element_type=jnp.float32)
            col = slice(h * N_CHUNK, (h + 1) * N_CHUNK)
            if k == 0:
                out_ref[:, col] = prod
            else:
                out_ref[:, col] = out_ref[:, col] + prod

        amax_local = jnp.max(jnp.abs(out_ref[:, 0:N_CHUNK]))
        for c in range(1, NCH):
            amax_local = jnp.maximum(
                amax_local,
                jnp.max(jnp.abs(out_ref[:, c * N_CHUNK:(c + 1) * N_CHUNK])),
            )
        amax_send[...] = jnp.full((8, 128), amax_local, jnp.float32)

        def amax_send_desc(j):
            return pltpu.make_async_remote_copy(
                src_ref=amax_send,
                dst_ref=amax_recv.at[my],
                send_sem=amax_send_sems.at[j],
                recv_sem=amax_recv_sems.at[my],
                device_id=(j,),
                device_id_type=pl.DeviceIdType.MESH,
            )

        def amax_recv_desc(k):
            return pltpu.make_async_remote_copy(
                src_ref=amax_send,
                dst_ref=amax_recv.at[k],
                send_sem=amax_send_sems.at[k],
                recv_sem=amax_recv_sems.at[k],
                device_id=(k,),
                device_id_type=pl.DeviceIdType.MESH,
            )

        for j in range(N_DEV):
            @pl.when(j != my)
            def _():
                amax_send_desc(j).start()
        for k in range(N_DEV):
            @pl.when(k != my)
            def _():
                amax_recv_desc(k).wait_recv()

        g_amax = jnp.maximum(jnp.max(amax_recv[...]), amax_local)

        scale = g_amax / 127.0
        for c in range(NCH):
            col = slice(c * N_CHUNK, (c + 1) * N_CHUNK)
            y = out_ref[:, col]
            q = jnp.clip(jnp.round(y / scale), -127.0, 127.0)
            out_ref[:, col] = q * scale

        for j in range(N_DEV):
            @pl.when(j != my)
            def _():
                a2a_send_desc(j).wait_send()
                amax_send_desc(j).wait_send()

    return pl.pallas_call(
        body,
        out_shape=jax.ShapeDtypeStruct((M_PER, N_OUT), jnp.float32),
        in_specs=[
            pl.BlockSpec(memory_space=pltpu.MemorySpace.HBM),
            pl.BlockSpec(memory_space=pltpu.MemorySpace.HBM),
        ],
        out_specs=pl.BlockSpec(memory_space=pltpu.MemorySpace.VMEM),
        scratch_shapes=[
            pltpu.VMEM((N_DEV, M_PER, K_PER), jnp.float32),
            pltpu.VMEM((2, K_PER, N_CHUNK), jnp.float32),
            pltpu.VMEM((8, 128), jnp.float32),
            pltpu.VMEM((N_DEV, 8, 128), jnp.float32),
            pltpu.SemaphoreType.DMA((N_DEV,)),
            pltpu.SemaphoreType.DMA((N_DEV,)),
            pltpu.SemaphoreType.DMA((N_DEV,)),
            pltpu.SemaphoreType.DMA((N_DEV,)),
            pltpu.SemaphoreType.DMA((2,)),
            pltpu.SemaphoreType.DMA(()),
        ],
        compiler_params=pltpu.CompilerParams(
            collective_id=0,
            vmem_limit_bytes=60 * 1024 * 1024,
        ),
    )(x, w_mat)


# device time: 139719 ns/iter; 1.1479x vs baseline; 1.1479x over previous
import os

import jax
import jax.numpy as jnp
from jax import lax
from jax.experimental import pallas as pl
from jax.experimental.pallas import tpu as pltpu

_VARIANT = os.environ.get("SCBAND_VARIANT", "")

N_DEV = 8
M = 4096
K = 4096
N_OUT = 8192
M_PER = M // N_DEV
K_PER = K // N_DEV
N_CHUNK = 2048
NCH = N_OUT // N_CHUNK


def kernel(x, w_mat):
    assert x.shape == (M, K_PER), x.shape
    assert w_mat.shape == (K, N_OUT), w_mat.shape

    def body(x_ref, w_ref, out_ref,
             gathered, wbuf, amax_send, amax_recv,
             a2a_send_sems, a2a_recv_sems,
             amax_send_sems, amax_recv_sems, w_sems, local_sem):
        my = lax.axis_index("i")

        amax_recv[...] = jnp.zeros_like(amax_recv)

        if _VARIANT != "noa2a":
            bar = pltpu.get_barrier_semaphore()
            for j in range(N_DEV):
                @pl.when(j != my)
                def _():
                    pl.semaphore_signal(
                        bar, inc=1, device_id=(j,),
                        device_id_type=pl.DeviceIdType.MESH,
                    )
            pl.semaphore_wait(bar, N_DEV - 1)

        def a2a_send_desc(j):
            return pltpu.make_async_remote_copy(
                src_ref=x_ref.at[pl.ds(j * M_PER, M_PER), :],
                dst_ref=gathered.at[my],
                send_sem=a2a_send_sems.at[j],
                recv_sem=a2a_recv_sems.at[my],
                device_id=(j,),
                device_id_type=pl.DeviceIdType.MESH,
            )

        def a2a_recv_desc(k):
            return pltpu.make_async_remote_copy(
                src_ref=gathered.at[k],
                dst_ref=gathered.at[k],
                send_sem=a2a_send_sems.at[k],
                recv_sem=a2a_recv_sems.at[k],
                device_id=(k,),
                device_id_type=pl.DeviceIdType.MESH,
            )

        if _VARIANT != "noa2a":
            for j in range(N_DEV):
                @pl.when(j != my)
                def _():
                    a2a_send_desc(j).start()

        def local_desc():
            return pltpu.make_async_copy(
                x_ref.at[pl.ds(my * M_PER, M_PER), :],
                gathered.at[my],
                local_sem,
            )

        if _VARIANT != "noa2a":
            local_desc().start()

        NW = N_DEV * NCH

        def w_desc(widx, slot):
            k, h = widx // NCH, widx % NCH
            return pltpu.make_async_copy(
                w_ref.at[pl.ds(k * K_PER, K_PER),
                         pl.ds(h * N_CHUNK, N_CHUNK)],
                wbuf.at[slot],
                w_sems.at[slot],
            )

        w_desc(0, 0).start()
        for widx in range(NW):
            k, h = widx // NCH, widx % NCH
            if h == 0:
                if _VARIANT == "noa2a":
                    d = pltpu.make_async_copy(
                        x_ref.at[pl.ds(k * M_PER, M_PER), :],
                        gathered.at[k], local_sem)
                    d.start()
                    d.wait()
                else:
                    @pl.when(k == my)
                    def _():
                        local_desc().wait()

                    @pl.when(k != my)
                    def _():
                        a2a_recv_desc(k).wait_recv()

            if widx + 1 < NW:
                w_desc(widx + 1, (widx + 1) % 2).start()
            w_desc(widx, widx % 2).wait()
            col = slice(h * N_CHUNK, (h + 1) * N_CHUNK)
            if _VARIANT == "nogemm":
                out_ref[:, col] = wbuf[widx % 2]
                continue
            if _VARIANT == "bf16":
                prod = jnp.dot(gathered[k].astype(jnp.bfloat16),
                               wbuf[widx % 2].astype(jnp.bfloat16),
                               preferred_element_type=jnp.float32)
            else:
                prod = jnp.dot(gathered[k], wbuf[widx % 2],
                               preferred_element_type=jnp.float32)
            if k == 0:
                out_ref[:, col] = prod
            else:
                out_ref[:, col] = out_ref[:, col] + prod

        amax_local = jnp.max(jnp.abs(out_ref[:, 0:N_CHUNK]))
        for c in range(1, NCH):
            amax_local = jnp.maximum(
                amax_local,
                jnp.max(jnp.abs(out_ref[:, c * N_CHUNK:(c + 1) * N_CHUNK])),
            )
        amax_send[...] = jnp.full((8, 128), amax_local, jnp.float32)

        def amax_send_desc(j):
            return pltpu.make_async_remote_copy(
                src_ref=amax_send,
                dst_ref=amax_recv.at[my],
                send_sem=amax_send_sems.at[j],
                recv_sem=amax_recv_sems.at[my],
                device_id=(j,),
                device_id_type=pl.DeviceIdType.MESH,
            )

        def amax_recv_desc(k):
            return pltpu.make_async_remote_copy(
                src_ref=amax_send,
                dst_ref=amax_recv.at[k],
                send_sem=amax_send_sems.at[k],
                recv_sem=amax_recv_sems.at[k],
                device_id=(k,),
                device_id_type=pl.DeviceIdType.MESH,
            )

        if _VARIANT != "noa2a":
            for j in range(N_DEV):
                @pl.when(j != my)
                def _():
                    amax_send_desc(j).start()
            for k in range(N_DEV):
                @pl.when(k != my)
                def _():
                    amax_recv_desc(k).wait_recv()

        g_amax = jnp.maximum(jnp.max(amax_recv[...]), amax_local)

        scale = g_amax / 127.0
        for c in range(NCH):
            col = slice(c * N_CHUNK, (c + 1) * N_CHUNK)
            y = out_ref[:, col]
            q = jnp.clip(jnp.round(y / scale), -127.0, 127.0)
            out_ref[:, col] = q * scale

        if _VARIANT != "noa2a":
            for j in range(N_DEV):
                @pl.when(j != my)
                def _():
                    a2a_send_desc(j).wait_send()
                    amax_send_desc(j).wait_send()

    return pl.pallas_call(
        body,
        out_shape=jax.ShapeDtypeStruct((M_PER, N_OUT), jnp.float32),
        in_specs=[
            pl.BlockSpec(memory_space=pltpu.MemorySpace.HBM),
            pl.BlockSpec(memory_space=pltpu.MemorySpace.HBM),
        ],
        out_specs=pl.BlockSpec(memory_space=pltpu.MemorySpace.VMEM),
        scratch_shapes=[
            pltpu.VMEM((N_DEV, M_PER, K_PER), jnp.float32),
            pltpu.VMEM((2, K_PER, N_CHUNK), jnp.float32),
            pltpu.VMEM((8, 128), jnp.float32),
            pltpu.VMEM((N_DEV, 8, 128), jnp.float32),
            pltpu.SemaphoreType.DMA((N_DEV,)),
            pltpu.SemaphoreType.DMA((N_DEV,)),
            pltpu.SemaphoreType.DMA((N_DEV,)),
            pltpu.SemaphoreType.DMA((N_DEV,)),
            pltpu.SemaphoreType.DMA((2,)),
            pltpu.SemaphoreType.DMA(()),
        ],
        compiler_params=pltpu.CompilerParams(
            collective_id=None if _VARIANT == "noa2a" else 0,
            vmem_limit_bytes=60 * 1024 * 1024,
        ),
    )(x, w_mat)


# device time: 113174 ns/iter; 1.4171x vs baseline; 1.2346x over previous
import jax
import jax.numpy as jnp
from jax import lax
from jax.experimental import pallas as pl
from jax.experimental.pallas import tpu as pltpu

N_DEV = 8
M = 4096
K = 4096
N_OUT = 8192
M_PER = M // N_DEV
K_PER = K // N_DEV
N_CHUNK = 2048
NCH = N_OUT // N_CHUNK
G = N_DEV // 2

_MESH = pl.DeviceIdType.MESH


def kernel(x, w_mat):
    assert x.shape == (M, K_PER), x.shape
    assert w_mat.shape == (K, N_OUT), w_mat.shape

    def body(x_ref, w_ref, out_ref,
             gathered, wbuf, amax_send, amax_recv,
             send_sems, recv_sems,
             amax_send_sems, amax_recv_sems, w_sems, local_sem):
        my = lax.axis_index("i")

        amax_recv[...] = jnp.zeros_like(amax_recv)

        bar = pltpu.get_barrier_semaphore()
        for j in range(N_DEV):
            @pl.when(j != my)
            def _():
                pl.semaphore_signal(
                    bar, inc=1, device_id=(j,), device_id_type=_MESH,
                )
        pl.semaphore_wait(bar, N_DEV - 1)

        def send_desc(s):
            tgt = lax.rem(my + (N_DEV - s), N_DEV)
            return pltpu.make_async_remote_copy(
                src_ref=x_ref.at[pl.ds(tgt * M_PER, M_PER), :],
                dst_ref=gathered.at[:, pl.ds(s * K_PER, K_PER)],
                send_sem=send_sems.at[s],
                recv_sem=recv_sems.at[s],
                device_id=(tgt,),
                device_id_type=_MESH,
            )

        def recv_desc(t):
            return pltpu.make_async_remote_copy(
                src_ref=gathered.at[:, pl.ds(t * K_PER, K_PER)],
                dst_ref=gathered.at[:, pl.ds(t * K_PER, K_PER)],
                send_sem=send_sems.at[t],
                recv_sem=recv_sems.at[t],
                device_id=(0,),
                device_id_type=_MESH,
            )

        for s in range(1, N_DEV):
            send_desc(s).start()

        def local_desc():
            return pltpu.make_async_copy(
                x_ref.at[pl.ds(my * M_PER, M_PER), :],
                gathered.at[:, pl.ds(0, K_PER)],
                local_sem,
            )

        local_desc().start()

        def w_descs(q, slot):
            g, c = q // NCH, q % NCH
            ds = []
            for u in range(2):
                t = 2 * g + u
                k = lax.rem(my + t, N_DEV)
                ds.append(pltpu.make_async_copy(
                    w_ref.at[pl.ds(k * K_PER, K_PER),
                             pl.ds(c * N_CHUNK, N_CHUNK)],
                    wbuf.at[slot, pl.ds(u * K_PER, K_PER), :],
                    w_sems.at[slot],
                ))
            return ds

        for d in w_descs(0, 0):
            d.start()

        amax_local = jnp.float32(0.0)
        NQ = G * NCH
        for q in range(NQ):
            g, c = q // NCH, q % NCH
            if c == 0:
                for t in (2 * g, 2 * g + 1):
                    if t == 0:
                        local_desc().wait()
                    else:
                        recv_desc(t).wait_recv()
            if q + 1 < NQ:
                for d in w_descs(q + 1, (q + 1) % 2):
                    d.start()
            for d in w_descs(q, q % 2):
                d.wait()
            xg = gathered[:, 2 * g * K_PER:(2 * g + 2) * K_PER]
            prod = jnp.dot(xg, wbuf[q % 2],
                           preferred_element_type=jnp.float32)
            col = slice(c * N_CHUNK, (c + 1) * N_CHUNK)
            if g == 0:
                res = prod
            else:
                res = out_ref[:, col] + prod
            if g == G - 1:
                amax_local = jnp.maximum(amax_local, jnp.max(jnp.abs(res)))
            out_ref[:, col] = res

        amax_send[...] = jnp.full((8, 128), amax_local, jnp.float32)

        def amax_send_desc(j):
            return pltpu.make_async_remote_copy(
                src_ref=amax_send,
                dst_ref=amax_recv.at[my],
                send_sem=amax_send_sems.at[j],
                recv_sem=amax_recv_sems.at[my],
                device_id=(j,),
                device_id_type=_MESH,
            )

        def amax_recv_desc(k):
            return pltpu.make_async_remote_copy(
                src_ref=amax_send,
                dst_ref=amax_recv.at[k],
                send_sem=amax_send_sems.at[k],
                recv_sem=amax_recv_sems.at[k],
                device_id=(k,),
                device_id_type=_MESH,
            )

        for j in range(N_DEV):
            @pl.when(j != my)
            def _():
                amax_send_desc(j).start()
        for k in range(N_DEV):
            @pl.when(k != my)
            def _():
                amax_recv_desc(k).wait_recv()

        g_amax = jnp.maximum(jnp.max(amax_recv[...]), amax_local)

        scale = g_amax / 127.0
        for c in range(NCH):
            col = slice(c * N_CHUNK, (c + 1) * N_CHUNK)
            y = out_ref[:, col]
            q8 = jnp.clip(jnp.round(y / scale), -127.0, 127.0)
            out_ref[:, col] = q8 * scale

        for s in range(1, N_DEV):
            send_desc(s).wait_send()
        for j in range(N_DEV):
            @pl.when(j != my)
            def _():
                amax_send_desc(j).wait_send()

    return pl.pallas_call(
        body,
        out_shape=jax.ShapeDtypeStruct((M_PER, N_OUT), jnp.float32),
        in_specs=[
            pl.BlockSpec(memory_space=pltpu.MemorySpace.HBM),
            pl.BlockSpec(memory_space=pltpu.MemorySpace.HBM),
        ],
        out_specs=pl.BlockSpec(memory_space=pltpu.MemorySpace.VMEM),
        scratch_shapes=[
            pltpu.VMEM((M_PER, K), jnp.float32),
            pltpu.VMEM((2, 2 * K_PER, N_CHUNK), jnp.float32),
            pltpu.VMEM((8, 128), jnp.float32),
            pltpu.VMEM((N_DEV, 8, 128), jnp.float32),
            pltpu.SemaphoreType.DMA((N_DEV,)),
            pltpu.SemaphoreType.DMA((N_DEV,)),
            pltpu.SemaphoreType.DMA((N_DEV,)),
            pltpu.SemaphoreType.DMA((N_DEV,)),
            pltpu.SemaphoreType.DMA((2,)),
            pltpu.SemaphoreType.DMA(()),
        ],
        compiler_params=pltpu.CompilerParams(
            collective_id=0,
            vmem_limit_bytes=60 * 1024 * 1024,
        ),
    )(x, w_mat)


# device time: 91096 ns/iter; 1.7606x vs baseline; 1.2424x over previous
import contextlib
import os

import jax
import jax.numpy as jnp
from jax import lax
from jax.experimental import pallas as pl
from jax.experimental.pallas import tpu as pltpu

_SCOPES = os.environ.get("SCBAND_SCOPES", "") == "1"
_VARIANT = os.environ.get("SCBAND_VARIANT", "")


def _scope(name):
    return jax.named_scope(name) if _SCOPES else contextlib.nullcontext()

N_DEV = 8
M = 4096
K = 4096
N_OUT = 8192
M_PER = M // N_DEV
K_PER = K // N_DEV
N_CHUNK = 2048
NCH = N_OUT // N_CHUNK
G = N_DEV // 2

_MESH = pl.DeviceIdType.MESH


def kernel(x, w_mat):
    assert x.shape == (M, K_PER), x.shape
    assert w_mat.shape == (K, N_OUT), w_mat.shape

    def body(x_ref, w_ref, out_ref,
             xsend, gathered, wbuf, amax_send, amax_recv,
             send_sems, recv_sems,
             amax_send_sems, amax_recv_sems, w_sems, local_sem):
        my = lax.axis_index("i")

        amax_recv[...] = jnp.zeros_like(amax_recv)

        xsend[...] = x_ref[...].astype(jnp.bfloat16)

        with _scope("barrier"):
            bar = pltpu.get_barrier_semaphore()
            for j in range(N_DEV):
                @pl.when(j != my)
                def _():
                    pl.semaphore_signal(
                        bar, inc=1, device_id=(j,), device_id_type=_MESH,
                    )
            pl.semaphore_wait(bar, N_DEV - 1)

        def send_desc(s):
            tgt = lax.rem(my + (N_DEV - s), N_DEV)
            return pltpu.make_async_remote_copy(
                src_ref=xsend.at[pl.ds(tgt * M_PER, M_PER), :],
                dst_ref=gathered.at[:, pl.ds(s * K_PER, K_PER)],
                send_sem=send_sems.at[s],
                recv_sem=recv_sems.at[s],
                device_id=(tgt,),
                device_id_type=_MESH,
            )

        def recv_desc(t):
            return pltpu.make_async_remote_copy(
                src_ref=gathered.at[:, pl.ds(t * K_PER, K_PER)],
                dst_ref=gathered.at[:, pl.ds(t * K_PER, K_PER)],
                send_sem=send_sems.at[t],
                recv_sem=recv_sems.at[t],
                device_id=(0,),
                device_id_type=_MESH,
            )

        for s in range(1, N_DEV):
            send_desc(s).start()

        def local_desc():
            return pltpu.make_async_copy(
                xsend.at[pl.ds(my * M_PER, M_PER), :],
                gathered.at[:, pl.ds(0, K_PER)],
                local_sem,
            )

        local_desc().start()

        if _VARIANT == "a2aonly":
            local_desc().wait()
            for t in range(1, N_DEV):
                recv_desc(t).wait_recv()
            out_ref[0:8, 0:128] = gathered[0:8, 0:128].astype(jnp.float32)
            for s in range(1, N_DEV):
                send_desc(s).wait_send()
            return

        def w_descs(q, slot):
            g, c = q // NCH, q % NCH
            ds = []
            for u in range(2):
                t = 2 * g + u
                k = lax.rem(my + t, N_DEV)
                ds.append(pltpu.make_async_copy(
                    w_ref.at[pl.ds(k * K_PER, K_PER),
                             pl.ds(c * N_CHUNK, N_CHUNK)],
                    wbuf.at[slot, pl.ds(u * K_PER, K_PER), :],
                    w_sems.at[slot],
                ))
            return ds

        for d in w_descs(0, 0):
            d.start()

        amax_local = jnp.float32(0.0)
        NQ = G * NCH
        for q in range(NQ):
            g, c = q // NCH, q % NCH
            if c == 0:
                with _scope(f"recv#g={g}"):
                    for t in (2 * g, 2 * g + 1):
                        if t == 0:
                            local_desc().wait()
                        else:
                            recv_desc(t).wait_recv()
            if q + 1 < NQ:
                for d in w_descs(q + 1, (q + 1) % 2):
                    d.start()
            with _scope(f"wwait#q={q}"):
                for d in w_descs(q, q % 2):
                    d.wait()
            with _scope(f"mm#q={q}"):
                xg = gathered[:, 2 * g * K_PER:(2 * g + 2) * K_PER]
                prod = jnp.dot(xg.astype(jnp.float32), wbuf[q % 2],
                               preferred_element_type=jnp.float32)
                col = slice(c * N_CHUNK, (c + 1) * N_CHUNK)
                if g == 0:
                    res = prod
                else:
                    res = out_ref[:, col] + prod
                if g == G - 1:
                    amax_local = jnp.maximum(amax_local,
                                             jnp.max(jnp.abs(res)))
                out_ref[:, col] = res

        amax_send[...] = jnp.full((8, 128), amax_local, jnp.float32)

        def amax_send_desc(j):
            return pltpu.make_async_remote_copy(
                src_ref=amax_send,
                dst_ref=amax_recv.at[my],
                send_sem=amax_send_sems.at[j],
                recv_sem=amax_recv_sems.at[my],
                device_id=(j,),
                device_id_type=_MESH,
            )

        def amax_recv_desc(k):
            return pltpu.make_async_remote_copy(
                src_ref=amax_send,
                dst_ref=amax_recv.at[k],
                send_sem=amax_send_sems.at[k],
                recv_sem=amax_recv_sems.at[k],
                device_id=(k,),
                device_id_type=_MESH,
            )

        with _scope("amax_xchg"):
            for j in range(N_DEV):
                @pl.when(j != my)
                def _():
                    amax_send_desc(j).start()
            for k in range(N_DEV):
                @pl.when(k != my)
                def _():
                    amax_recv_desc(k).wait_recv()

            g_amax = jnp.maximum(jnp.max(amax_recv[...]), amax_local)

        with _scope("quant"):
            scale = g_amax / 127.0
            for c in range(NCH):
                col = slice(c * N_CHUNK, (c + 1) * N_CHUNK)
                y = out_ref[:, col]
                q8 = jnp.clip(jnp.round(y / scale), -127.0, 127.0)
                out_ref[:, col] = q8 * scale

        with _scope("drain"):
            for s in range(1, N_DEV):
                send_desc(s).wait_send()
            for j in range(N_DEV):
                @pl.when(j != my)
                def _():
                    amax_send_desc(j).wait_send()

    return pl.pallas_call(
        body,
        out_shape=jax.ShapeDtypeStruct((M_PER, N_OUT), jnp.float32),
        in_specs=[
            pl.BlockSpec(memory_space=pltpu.MemorySpace.VMEM),
            pl.BlockSpec(memory_space=pltpu.MemorySpace.HBM),
        ],
        out_specs=pl.BlockSpec(memory_space=pltpu.MemorySpace.VMEM),
        scratch_shapes=[
            pltpu.VMEM((M, K_PER), jnp.bfloat16),
            pltpu.VMEM((M_PER, K), jnp.bfloat16),
            pltpu.VMEM((2, 2 * K_PER, N_CHUNK), jnp.float32),
            pltpu.VMEM((8, 128), jnp.float32),
            pltpu.VMEM((N_DEV, 8, 128), jnp.float32),
            pltpu.SemaphoreType.DMA((N_DEV,)),
            pltpu.SemaphoreType.DMA((N_DEV,)),
            pltpu.SemaphoreType.DMA((N_DEV,)),
            pltpu.SemaphoreType.DMA((N_DEV,)),
            pltpu.SemaphoreType.DMA((2,)),
            pltpu.SemaphoreType.DMA(()),
        ],
        compiler_params=pltpu.CompilerParams(
            collective_id=0,
            vmem_limit_bytes=60 * 1024 * 1024,
        ),
    )(x, w_mat)


# device time: 90480 ns/iter; 1.7726x vs baseline; 1.0068x over previous
import contextlib
import os

import jax
import jax.numpy as jnp
from jax import lax
from jax.experimental import pallas as pl
from jax.experimental.pallas import tpu as pltpu

_SCOPES = os.environ.get("SCBAND_SCOPES", "") == "1"
_VARIANT = os.environ.get("SCBAND_VARIANT", "")


def _scope(name):
    return jax.named_scope(name) if _SCOPES else contextlib.nullcontext()


N_DEV = 8
M = 4096
K = 4096
N_OUT = 8192
M_PER = M // N_DEV
K_PER = K // N_DEV
N_CHUNK = 2048
NCH = N_OUT // N_CHUNK

GROUPS = ((0, 1), (2, 3), (4, 5), (6,), (7,))

_MESH = pl.DeviceIdType.MESH


def kernel(x, w_mat):
    assert x.shape == (M, K_PER), x.shape
    assert w_mat.shape == (K, N_OUT), w_mat.shape

    def body(x_ref, w_ref, out_ref,
             xsend, gathered, wbuf, acc, amax_send, amax_recv,
             send_sems, recv_sems,
             amax_send_sems, amax_recv_sems, w_sems, local_sem,
             out_sems):
        my = lax.axis_index("i")

        amax_recv[...] = jnp.zeros_like(amax_recv)

        xsend[...] = x_ref[...].astype(jnp.bfloat16)

        with _scope("barrier"):
            bar = pltpu.get_barrier_semaphore()
            for j in range(N_DEV):
                @pl.when(j != my)
                def _():
                    pl.semaphore_signal(
                        bar, inc=1, device_id=(j,), device_id_type=_MESH,
                    )
            pl.semaphore_wait(bar, N_DEV - 1)

        def send_desc(s):
            tgt = lax.rem(my + (N_DEV - s), N_DEV)
            return pltpu.make_async_remote_copy(
                src_ref=xsend.at[pl.ds(tgt * M_PER, M_PER), :],
                dst_ref=gathered.at[:, pl.ds(s * K_PER, K_PER)],
                send_sem=send_sems.at[s],
                recv_sem=recv_sems.at[s],
                device_id=(tgt,),
                device_id_type=_MESH,
            )

        def recv_desc(t):
            return pltpu.make_async_remote_copy(
                src_ref=gathered.at[:, pl.ds(t * K_PER, K_PER)],
                dst_ref=gathered.at[:, pl.ds(t * K_PER, K_PER)],
                send_sem=send_sems.at[t],
                recv_sem=recv_sems.at[t],
                device_id=(0,),
                device_id_type=_MESH,
            )

        for s in range(1, N_DEV):
            send_desc(s).start()

        def local_desc():
            return pltpu.make_async_copy(
                xsend.at[pl.ds(my * M_PER, M_PER), :],
                gathered.at[:, pl.ds(0, K_PER)],
                local_sem,
            )

        local_desc().start()

        if _VARIANT == "a2aonly":
            local_desc().wait()
            for t in range(1, N_DEV):
                recv_desc(t).wait_recv()
            acc[0:8, 0:128] = gathered[0:8, 0:128].astype(jnp.float32)
            d = pltpu.make_async_copy(
                acc.at[0:8, 0:128], out_ref.at[0:8, 0:128], out_sems.at[0])
            d.start()
            d.wait()
            for s in range(1, N_DEV):
                send_desc(s).wait_send()
            return

        NQ = len(GROUPS) * NCH

        def w_descs(q, slot):
            gi, c = q // NCH, q % NCH
            ds = []
            for u, t in enumerate(GROUPS[gi]):
                k = lax.rem(my + t, N_DEV)
                ds.append(pltpu.make_async_copy(
                    w_ref.at[pl.ds(k * K_PER, K_PER),
                             pl.ds(c * N_CHUNK, N_CHUNK)],
                    wbuf.at[slot, pl.ds(u * K_PER, K_PER), :],
                    w_sems.at[slot],
                ))
            return ds

        for d in w_descs(0, 0):
            d.start()

        amax_local = jnp.float32(0.0)
        for q in range(NQ):
            gi, c = q // NCH, q % NCH
            grp = GROUPS[gi]
            if c == 0:
                with _scope(f"recv#g={gi}"):
                    for t in grp:
                        if t == 0:
                            local_desc().wait()
                        else:
                            recv_desc(t).wait_recv()
            if q + 1 < NQ:
                for d in w_descs(q + 1, (q + 1) % 2):
                    d.start()
            with _scope(f"wwait#q={q}"):
                for d in w_descs(q, q % 2):
                    d.wait()
            with _scope(f"mm#q={q}"):
                t0, t1 = grp[0], grp[-1] + 1
                xg = gathered[:, t0 * K_PER:t1 * K_PER]
                wg = wbuf[q % 2] if len(grp) == 2 else wbuf[q % 2, 0:K_PER, :]
                prod = jnp.dot(xg.astype(jnp.float32), wg,
                               preferred_element_type=jnp.float32)
                col = slice(c * N_CHUNK, (c + 1) * N_CHUNK)
                if gi == 0:
                    res = prod
                else:
                    res = acc[:, col] + prod
                if gi == len(GROUPS) - 1:
                    amax_local = jnp.maximum(amax_local,
                                             jnp.max(jnp.abs(res)))
                acc[:, col] = res

        amax_send[...] = jnp.full((8, 128), amax_local, jnp.float32)

        def amax_send_desc(j):
            return pltpu.make_async_remote_copy(
                src_ref=amax_send,
                dst_ref=amax_recv.at[my],
                send_sem=amax_send_sems.at[j],
                recv_sem=amax_recv_sems.at[my],
                device_id=(j,),
                device_id_type=_MESH,
            )

        def amax_recv_desc(k):
            return pltpu.make_async_remote_copy(
                src_ref=amax_send,
                dst_ref=amax_recv.at[k],
                send_sem=amax_send_sems.at[k],
                recv_sem=amax_recv_sems.at[k],
                device_id=(k,),
                device_id_type=_MESH,
            )

        with _scope("amax_xchg"):
            for j in range(N_DEV):
                @pl.when(j != my)
                def _():
                    amax_send_desc(j).start()
            for k in range(N_DEV):
                @pl.when(k != my)
                def _():
                    amax_recv_desc(k).wait_recv()

            g_amax = jnp.maximum(jnp.max(amax_recv[...]), amax_local)

        with _scope("quant"):
            scale = g_amax / 127.0
            for c in range(NCH):
                col = slice(c * N_CHUNK, (c + 1) * N_CHUNK)
                y = acc[:, col]
                q8 = jnp.clip(jnp.round(y / scale), -127.0, 127.0)
                acc[:, col] = q8 * scale
                pltpu.make_async_copy(
                    acc.at[:, col], out_ref.at[:, col], out_sems.at[c],
                ).start()
            for c in range(NCH):
                col = slice(c * N_CHUNK, (c + 1) * N_CHUNK)
                pltpu.make_async_copy(
                    acc.at[:, col], out_ref.at[:, col], out_sems.at[c],
                ).wait()

        with _scope("drain"):
            for s in range(1, N_DEV):
                send_desc(s).wait_send()
            for j in range(N_DEV):
                @pl.when(j != my)
                def _():
                    amax_send_desc(j).wait_send()

    return pl.pallas_call(
        body,
        out_shape=jax.ShapeDtypeStruct((M_PER, N_OUT), jnp.float32),
        in_specs=[
            pl.BlockSpec(memory_space=pltpu.MemorySpace.VMEM),
            pl.BlockSpec(memory_space=pltpu.MemorySpace.HBM),
        ],
        out_specs=pl.BlockSpec(memory_space=pltpu.MemorySpace.HBM),
        scratch_shapes=[
            pltpu.VMEM((M, K_PER), jnp.bfloat16),
            pltpu.VMEM((M_PER, K), jnp.bfloat16),
            pltpu.VMEM((2, 2 * K_PER, N_CHUNK), jnp.float32),
            pltpu.VMEM((M_PER, N_OUT), jnp.float32),
            pltpu.VMEM((8, 128), jnp.float32),
            pltpu.VMEM((N_DEV, 8, 128), jnp.float32),
            pltpu.SemaphoreType.DMA((N_DEV,)),
            pltpu.SemaphoreType.DMA((N_DEV,)),
            pltpu.SemaphoreType.DMA((N_DEV,)),
            pltpu.SemaphoreType.DMA((N_DEV,)),
            pltpu.SemaphoreType.DMA((2,)),
            pltpu.SemaphoreType.DMA(()),
            pltpu.SemaphoreType.DMA((NCH,)),
        ],
        compiler_params=pltpu.CompilerParams(
            collective_id=0,
            vmem_limit_bytes=60 * 1024 * 1024,
        ),
    )(x, w_mat)


# device time: 81468 ns/iter; 1.9687x vs baseline; 1.1106x over previous
import contextlib
import os

import jax
import jax.numpy as jnp
from jax import lax
from jax.experimental import pallas as pl
from jax.experimental.pallas import tpu as pltpu

_SCOPES = os.environ.get("SCBAND_SCOPES", "") == "1"
_VARIANT = os.environ.get("SCBAND_VARIANT", "")


def _scope(name):
    return jax.named_scope(name) if _SCOPES else contextlib.nullcontext()


N_DEV = 8
M = 4096
K = 4096
N_OUT = 8192
M_PER = M // N_DEV
K_PER = K // N_DEV
N_CHUNK = 2048
NCH = N_OUT // N_CHUNK

GROUPS = ((0, 1), (2, 3), (4, 5), (6,), (7,))

_MESH = pl.DeviceIdType.MESH


def kernel(x, w_mat):
    assert x.shape == (M, K_PER), x.shape
    assert w_mat.shape == (K, N_OUT), w_mat.shape

    def body(x_ref, w_ref, out_ref,
             xsend, gathered, wbuf, acc, amax_send, amax_recv,
             send_sems, recv_sems,
             amax_send_sems, amax_recv_sems, w_sems, local_sem,
             out_sems):
        my = lax.axis_index("i")

        amax_recv[...] = jnp.zeros_like(amax_recv)

        xsend[...] = x_ref[...].astype(jnp.bfloat16)

        with _scope("barrier"):
            bar = pltpu.get_barrier_semaphore()
            for j in range(N_DEV):
                @pl.when(j != my)
                def _():
                    pl.semaphore_signal(
                        bar, inc=1, device_id=(j,), device_id_type=_MESH,
                    )
            pl.semaphore_wait(bar, N_DEV - 1)

        def send_desc(s):
            tgt = lax.rem(my + (N_DEV - s), N_DEV)
            return pltpu.make_async_remote_copy(
                src_ref=xsend.at[pl.ds(tgt * M_PER, M_PER), :],
                dst_ref=gathered.at[:, pl.ds(s * K_PER, K_PER)],
                send_sem=send_sems.at[s],
                recv_sem=recv_sems.at[s],
                device_id=(tgt,),
                device_id_type=_MESH,
            )

        def recv_desc(t):
            return pltpu.make_async_remote_copy(
                src_ref=gathered.at[:, pl.ds(t * K_PER, K_PER)],
                dst_ref=gathered.at[:, pl.ds(t * K_PER, K_PER)],
                send_sem=send_sems.at[t],
                recv_sem=recv_sems.at[t],
                device_id=(0,),
                device_id_type=_MESH,
            )

        for s in range(1, N_DEV):
            send_desc(s).start()

        def local_desc():
            return pltpu.make_async_copy(
                xsend.at[pl.ds(my * M_PER, M_PER), :],
                gathered.at[:, pl.ds(0, K_PER)],
                local_sem,
            )

        local_desc().start()

        if _VARIANT == "a2aonly":
            local_desc().wait()
            for t in range(1, N_DEV):
                recv_desc(t).wait_recv()
            acc[0:8, 0:128] = gathered[0:8, 0:128].astype(jnp.float32)
            d = pltpu.make_async_copy(
                acc.at[0:8, 0:128], out_ref.at[0:8, 0:128], out_sems.at[0])
            d.start()
            d.wait()
            for s in range(1, N_DEV):
                send_desc(s).wait_send()
            return

        NQ = len(GROUPS) * NCH
        NB = 3

        def w_descs(q, slot):
            gi, c = q // NCH, q % NCH
            ds = []
            for u, t in enumerate(GROUPS[gi]):
                k = lax.rem(my + t, N_DEV)
                ds.append(pltpu.make_async_copy(
                    w_ref.at[pl.ds(k * K_PER, K_PER),
                             pl.ds(c * N_CHUNK, N_CHUNK)],
                    wbuf.at[slot, pl.ds(u * K_PER, K_PER), :],
                    w_sems.at[slot],
                ))
            return ds

        for q0 in range(NB - 1):
            for d in w_descs(q0, q0 % NB):
                d.start()

        amax_local = jnp.float32(0.0)
        for q in range(NQ):
            gi, c = q // NCH, q % NCH
            grp = GROUPS[gi]
            if q + NB - 1 < NQ:
                for d in w_descs(q + NB - 1, (q + NB - 1) % NB):
                    d.start()
            if c == 0:
                with _scope(f"recv#g={gi}"):
                    for t in grp:
                        if t == 0:
                            local_desc().wait()
                        else:
                            recv_desc(t).wait_recv()
            with _scope(f"wwait#q={q}"):
                for d in w_descs(q, q % NB):
                    d.wait()
            with _scope(f"mm#q={q}"):
                t0, t1 = grp[0], grp[-1] + 1
                xg = gathered[:, t0 * K_PER:t1 * K_PER]
                wg = (wbuf[q % NB] if len(grp) == 2
                      else wbuf[q % NB, 0:K_PER, :])
                prod = jnp.dot(xg.astype(jnp.float32), wg,
                               preferred_element_type=jnp.float32)
                col = slice(c * N_CHUNK, (c + 1) * N_CHUNK)
                if gi == 0:
                    res = prod
                else:
                    res = acc[:, col] + prod
                if gi == len(GROUPS) - 1:
                    amax_local = jnp.maximum(amax_local,
                                             jnp.max(jnp.abs(res)))
                acc[:, col] = res

        amax_send[...] = jnp.full((8, 128), amax_local, jnp.float32)

        def amax_send_desc(j):
            return pltpu.make_async_remote_copy(
                src_ref=amax_send,
                dst_ref=amax_recv.at[my],
                send_sem=amax_send_sems.at[j],
                recv_sem=amax_recv_sems.at[my],
                device_id=(j,),
                device_id_type=_MESH,
            )

        def amax_recv_desc(k):
            return pltpu.make_async_remote_copy(
                src_ref=amax_send,
                dst_ref=amax_recv.at[k],
                send_sem=amax_send_sems.at[k],
                recv_sem=amax_recv_sems.at[k],
                device_id=(k,),
                device_id_type=_MESH,
            )

        with _scope("amax_xchg"):
            for j in range(N_DEV):
                @pl.when(j != my)
                def _():
                    amax_send_desc(j).start()
            for k in range(N_DEV):
                @pl.when(k != my)
                def _():
                    amax_recv_desc(k).wait_recv()

            g_amax = jnp.maximum(jnp.max(amax_recv[...]), amax_local)

        with _scope("quant"):
            scale = g_amax / 127.0
            for c in range(NCH):
                col = slice(c * N_CHUNK, (c + 1) * N_CHUNK)
                y = acc[:, col]
                q8 = jnp.clip(jnp.round(y / scale), -127.0, 127.0)
                acc[:, col] = q8 * scale
                pltpu.make_async_copy(
                    acc.at[:, col], out_ref.at[:, col], out_sems.at[c],
                ).start()
            for c in range(NCH):
                col = slice(c * N_CHUNK, (c + 1) * N_CHUNK)
                pltpu.make_async_copy(
                    acc.at[:, col], out_ref.at[:, col], out_sems.at[c],
                ).wait()

        with _scope("drain"):
            for s in range(1, N_DEV):
                send_desc(s).wait_send()
            for j in range(N_DEV):
                @pl.when(j != my)
                def _():
                    amax_send_desc(j).wait_send()

    return pl.pallas_call(
        body,
        out_shape=jax.ShapeDtypeStruct((M_PER, N_OUT), jnp.float32),
        in_specs=[
            pl.BlockSpec(memory_space=pltpu.MemorySpace.VMEM),
            pl.BlockSpec(memory_space=pltpu.MemorySpace.HBM),
        ],
        out_specs=pl.BlockSpec(memory_space=pltpu.MemorySpace.HBM),
        scratch_shapes=[
            pltpu.VMEM((M, K_PER), jnp.bfloat16),
            pltpu.VMEM((M_PER, K), jnp.bfloat16),
            pltpu.VMEM((3, 2 * K_PER, N_CHUNK), jnp.float32),
            pltpu.VMEM((M_PER, N_OUT), jnp.float32),
            pltpu.VMEM((8, 128), jnp.float32),
            pltpu.VMEM((N_DEV, 8, 128), jnp.float32),
            pltpu.SemaphoreType.DMA((N_DEV,)),
            pltpu.SemaphoreType.DMA((N_DEV,)),
            pltpu.SemaphoreType.DMA((N_DEV,)),
            pltpu.SemaphoreType.DMA((N_DEV,)),
            pltpu.SemaphoreType.DMA((3,)),
            pltpu.SemaphoreType.DMA(()),
            pltpu.SemaphoreType.DMA((NCH,)),
        ],
        compiler_params=pltpu.CompilerParams(
            collective_id=0,
            vmem_limit_bytes=60 * 1024 * 1024,
        ),
    )(x, w_mat)


# device time: 80309 ns/iter; 1.9971x vs baseline; 1.0144x over previous
import contextlib
import os

import jax
import jax.numpy as jnp
from jax import lax
from jax.experimental import pallas as pl
from jax.experimental.pallas import tpu as pltpu

_SCOPES = os.environ.get("SCBAND_SCOPES", "") == "1"
_VARIANT = os.environ.get("SCBAND_VARIANT", "")


def _scope(name):
    return jax.named_scope(name) if _SCOPES else contextlib.nullcontext()


N_DEV = 8
M = 4096
K = 4096
N_OUT = 8192
M_PER = M // N_DEV
K_PER = K // N_DEV
N_CHUNK = 2048
NCH = N_OUT // N_CHUNK

GROUPS = ((0, 1), (2, 3), (4, 5), (6,), (7,))

_MESH = pl.DeviceIdType.MESH


def kernel(x, w_mat):
    assert x.shape == (M, K_PER), x.shape
    assert w_mat.shape == (K, N_OUT), w_mat.shape

    def body(x_ref, w_ref, out_ref,
             xsend, gathered, wbuf, acc, amax_send, amax_recv,
             send_sems, recv_sems,
             amax_send_sems, amax_recv_sems, w_sems, local_sem,
             out_sems):
        my = lax.axis_index("i")

        amax_recv[...] = jnp.zeros_like(amax_recv)

        xsend[...] = x_ref[...].astype(jnp.bfloat16)

        with _scope("barrier"):
            bar = pltpu.get_barrier_semaphore()
            for j in range(N_DEV):
                @pl.when(j != my)
                def _():
                    pl.semaphore_signal(
                        bar, inc=1, device_id=(j,), device_id_type=_MESH,
                    )
            pl.semaphore_wait(bar, N_DEV - 1)

        def send_desc(s):
            tgt = lax.rem(my + (N_DEV - s), N_DEV)
            return pltpu.make_async_remote_copy(
                src_ref=xsend.at[pl.ds(tgt * M_PER, M_PER), :],
                dst_ref=gathered.at[:, pl.ds(s * K_PER, K_PER)],
                send_sem=send_sems.at[s],
                recv_sem=recv_sems.at[s],
                device_id=(tgt,),
                device_id_type=_MESH,
            )

        def recv_desc(t):
            return pltpu.make_async_remote_copy(
                src_ref=gathered.at[:, pl.ds(t * K_PER, K_PER)],
                dst_ref=gathered.at[:, pl.ds(t * K_PER, K_PER)],
                send_sem=send_sems.at[t],
                recv_sem=recv_sems.at[t],
                device_id=(0,),
                device_id_type=_MESH,
            )

        for s in range(1, N_DEV):
            send_desc(s).start()

        def local_desc():
            return pltpu.make_async_copy(
                xsend.at[pl.ds(my * M_PER, M_PER), :],
                gathered.at[:, pl.ds(0, K_PER)],
                local_sem,
            )

        local_desc().start()

        if _VARIANT == "a2aonly":
            local_desc().wait()
            for t in range(1, N_DEV):
                recv_desc(t).wait_recv()
            acc[0:8, 0:128] = gathered[0:8, 0:128].astype(jnp.float32)
            d = pltpu.make_async_copy(
                acc.at[0:8, 0:128], out_ref.at[0:8, 0:128], out_sems.at[0])
            d.start()
            d.wait()
            for s in range(1, N_DEV):
                send_desc(s).wait_send()
            return

        NQ = len(GROUPS) * NCH
        NB = 3

        def w_descs(q, slot):
            gi, c = q // NCH, q % NCH
            ds = []
            for u, t in enumerate(GROUPS[gi]):
                k = lax.rem(my + t, N_DEV)
                ds.append(pltpu.make_async_copy(
                    w_ref.at[pl.ds(k * K_PER, K_PER),
                             pl.ds(c * N_CHUNK, N_CHUNK)],
                    wbuf.at[slot, pl.ds(u * K_PER, K_PER), :],
                    w_sems.at[slot],
                ))
            return ds

        for q0 in range(NB - 1):
            for d in w_descs(q0, q0 % NB):
                d.start()

        amax_local = jnp.float32(0.0)
        for q in range(NQ):
            gi, c = q // NCH, q % NCH
            grp = GROUPS[gi]
            if q + NB - 1 < NQ:
                for d in w_descs(q + NB - 1, (q + NB - 1) % NB):
                    d.start()
            if c == 0:
                with _scope(f"recv#g={gi}"):
                    for t in grp:
                        if t == 0:
                            local_desc().wait()
                        else:
                            recv_desc(t).wait_recv()
            with _scope(f"wwait#q={q}"):
                for d in w_descs(q, q % NB):
                    d.wait()
            with _scope(f"mm#q={q}"):
                t0, t1 = grp[0], grp[-1] + 1
                xg = gathered[:, t0 * K_PER:t1 * K_PER]
                wg = (wbuf[q % NB] if len(grp) == 2
                      else wbuf[q % NB, 0:K_PER, :])
                prod = jnp.dot(xg.astype(jnp.float32), wg,
                               preferred_element_type=jnp.float32)
                col = slice(c * N_CHUNK, (c + 1) * N_CHUNK)
                if gi == 0:
                    res = prod
                else:
                    res = acc[:, col] + prod
                if gi == len(GROUPS) - 1:
                    amax_local = jnp.maximum(amax_local,
                                             jnp.max(jnp.abs(res)))
                acc[:, col] = res

        amax_send[...] = jnp.full((8, 128), amax_local, jnp.float32)

        def amax_send_desc(j):
            return pltpu.make_async_remote_copy(
                src_ref=amax_send,
                dst_ref=amax_recv.at[my],
                send_sem=amax_send_sems.at[j],
                recv_sem=amax_recv_sems.at[my],
                device_id=(j,),
                device_id_type=_MESH,
            )

        def amax_recv_desc(k):
            return pltpu.make_async_remote_copy(
                src_ref=amax_send,
                dst_ref=amax_recv.at[k],
                send_sem=amax_send_sems.at[k],
                recv_sem=amax_recv_sems.at[k],
                device_id=(k,),
                device_id_type=_MESH,
            )

        with _scope("amax_xchg"):
            for j in range(N_DEV):
                @pl.when(j != my)
                def _():
                    amax_send_desc(j).start()
            for k in range(N_DEV):
                @pl.when(k != my)
                def _():
                    amax_recv_desc(k).wait_recv()

            g_amax = jnp.maximum(jnp.max(amax_recv[...]), amax_local)

        with _scope("quant"):
            scale = g_amax / 127.0
            inv = 127.0 / g_amax
            for c in range(NCH):
                col = slice(c * N_CHUNK, (c + 1) * N_CHUNK)
                y = acc[:, col]
                q8 = jnp.clip(jnp.round(y * inv), -127.0, 127.0)
                acc[:, col] = q8 * scale
                pltpu.make_async_copy(
                    acc.at[:, col], out_ref.at[:, col], out_sems.at[c],
                ).start()
            for c in range(NCH):
                col = slice(c * N_CHUNK, (c + 1) * N_CHUNK)
                pltpu.make_async_copy(
                    acc.at[:, col], out_ref.at[:, col], out_sems.at[c],
                ).wait()

        with _scope("drain"):
            for s in range(1, N_DEV):
                send_desc(s).wait_send()
            for j in range(N_DEV):
                @pl.when(j != my)
                def _():
                    amax_send_desc(j).wait_send()

    return pl.pallas_call(
        body,
        out_shape=jax.ShapeDtypeStruct((M_PER, N_OUT), jnp.float32),
        in_specs=[
            pl.BlockSpec(memory_space=pltpu.MemorySpace.VMEM),
            pl.BlockSpec(memory_space=pltpu.MemorySpace.HBM),
        ],
        out_specs=pl.BlockSpec(memory_space=pltpu.MemorySpace.HBM),
        scratch_shapes=[
            pltpu.VMEM((M, K_PER), jnp.bfloat16),
            pltpu.VMEM((M_PER, K), jnp.bfloat16),
            pltpu.VMEM((3, 2 * K_PER, N_CHUNK), jnp.float32),
            pltpu.VMEM((M_PER, N_OUT), jnp.float32),
            pltpu.VMEM((8, 128), jnp.float32),
            pltpu.VMEM((N_DEV, 8, 128), jnp.float32),
            pltpu.SemaphoreType.DMA((N_DEV,)),
            pltpu.SemaphoreType.DMA((N_DEV,)),
            pltpu.SemaphoreType.DMA((N_DEV,)),
            pltpu.SemaphoreType.DMA((N_DEV,)),
            pltpu.SemaphoreType.DMA((3,)),
            pltpu.SemaphoreType.DMA(()),
            pltpu.SemaphoreType.DMA((NCH,)),
        ],
        compiler_params=pltpu.CompilerParams(
            collective_id=0,
            vmem_limit_bytes=60 * 1024 * 1024,
        ),
    )(x, w_mat)
